# Initial kernel scaffold; baseline (speedup 1.0000x reference)
import jax
import jax.numpy as jnp
from jax.experimental import pallas as pl


def kernel(x, label, mask, edge_index, edge_weight, W1, W2):
    raise NotImplementedError("write your pallas kernel here")


# speedup vs baseline: 3.0330x; 3.0330x over previous
import functools

import jax
import jax.numpy as jnp
from jax import lax
from jax.experimental import pallas as pl
from jax.experimental.pallas import tpu as pltpu, tpu_sc as plsc

N_NODES = 10000
N_EDGES = 320000
D_IN = 128
D_HID = 128
D_OUT = 16
WEIGHT_DECAY = 5e-4

NC = 2
NS = 16
NW = NC * NS
E_PAD = 327680
EPW = E_PAD // NW
CHUNK = 64
NCHUNK = EPW // CHUNK
CPS = 32
NSUPER = NCHUNK // CPS
N_PAD = 10240
ROWS_PER_SUB = N_PAD // NS


@functools.lru_cache(maxsize=None)
def _make_spmm(d):
    mesh = plsc.VectorSubcoreMesh(core_axis_name="c", subcore_axis_name="s")

    @functools.partial(
        pl.kernel,
        out_type=jax.ShapeDtypeStruct((NC * N_PAD, d), jnp.float32),
        mesh=mesh,
        scratch_types=[
            pltpu.VMEM_SHARED((N_PAD, d), jnp.float32),
            pltpu.VMEM((CPS, CHUNK), jnp.int32),
            pltpu.VMEM((CPS, CHUNK), jnp.int32),
            pltpu.VMEM((CPS, CHUNK), jnp.float32),
            pltpu.VMEM((CHUNK, d), jnp.float32),
            pltpu.SemaphoreType.DMA,
        ],
    )
    def spmm(h_hbm, src_hbm, dst_hbm, w_hbm, zero_hbm, out_hbm,
             acc, src_v, dst_v, w_v, rows_v, sem):
        c = lax.axis_index("c")
        s = lax.axis_index("s")
        wid = s * NC + c
        pltpu.sync_copy(zero_hbm.at[pl.ds(s * ROWS_PER_SUB, ROWS_PER_SUB)],
                        acc.at[pl.ds(s * ROWS_PER_SUB, ROWS_PER_SUB)])
        plsc.subcore_barrier()

        def super_body(u, carry):
            pltpu.sync_copy(src_hbm.at[wid, pl.ds(u * CPS, CPS)], src_v)
            pltpu.sync_copy(dst_hbm.at[wid, pl.ds(u * CPS, CPS)], dst_v)
            pltpu.sync_copy(w_hbm.at[wid, pl.ds(u * CPS, CPS)], w_v)

            def chunk_body(k, carry2):
                pltpu.async_copy(h_hbm.at[src_v.at[k]], rows_v, sem).wait()
                for g in range(CHUNK // 16):
                    wvec = w_v[k, pl.ds(g * 16, 16)]
                    for j in range(16):
                        e = g * 16 + j
                        wsc = wvec[j]
                        for f in range(d // 16):
                            rows_v[e, pl.ds(f * 16, 16)] = (
                                rows_v[e, pl.ds(f * 16, 16)] * wsc)
                pltpu.sync_copy(rows_v, acc.at[dst_v.at[k]], add=True)
                return carry2

            lax.fori_loop(0, CPS, chunk_body, carry)
            return carry

        lax.fori_loop(0, NSUPER, super_body, 0)
        plsc.subcore_barrier()
        pltpu.sync_copy(acc.at[pl.ds(s * ROWS_PER_SUB, ROWS_PER_SUB)],
                        out_hbm.at[pl.ds(c * N_PAD + s * ROWS_PER_SUB,
                                         ROWS_PER_SUB)])

    return spmm


def _mm1_body(x_ref, w_ref, o_ref):
    o_ref[...] = jnp.dot(x_ref[...], w_ref[...],
                         preferred_element_type=jnp.float32)


def _relu_body(p0_ref, p1_ref, o_ref):
    o_ref[...] = jnp.maximum(p0_ref[...] + p1_ref[...], 0.0)


def _loss_body(o0_ref, o1_ref, lab_ref, m_ref, w1_ref, w2_ref, loss_ref,
               acc_ref):
    out = jnp.dot(o0_ref[...] + o1_ref[...], w2_ref[...],
                  preferred_element_type=jnp.float32)
    lab = lab_ref[...]
    m = m_ref[...]
    mx = jnp.max(out, axis=-1, keepdims=True)
    sh = out - mx
    lse = jnp.log(jnp.sum(jnp.exp(sh), axis=-1, keepdims=True))
    ce = -jnp.sum(lab * (sh - lse), axis=-1, keepdims=True)
    iota = lax.broadcasted_iota(jnp.int32, (N_NODES, D_OUT), 1)
    ao = jnp.min(jnp.where(out == mx, iota, D_OUT), axis=-1, keepdims=True)
    lmx = jnp.max(lab, axis=-1, keepdims=True)
    al = jnp.min(jnp.where(lab == lmx, iota, D_OUT), axis=-1, keepdims=True)
    correct = (ao == al).astype(jnp.float32)
    msum = jnp.sum(m)
    wd = WEIGHT_DECAY * 0.5 * jnp.sum(w1_ref[...] * w1_ref[...])
    loss_ref[...] = jnp.reshape(wd + jnp.sum(ce * m) / msum, (1, 1))
    acc_ref[...] = jnp.reshape(jnp.sum(correct * m) / msum, (1, 1))


def kernel(x, label, mask, edge_index, edge_weight, W1, W2):
    src = edge_index[0].astype(jnp.int32)
    dst = edge_index[1].astype(jnp.int32)
    pad = E_PAD - N_EDGES
    src = jnp.concatenate([src, jnp.zeros((pad,), jnp.int32)])
    dst = jnp.concatenate([dst, jnp.zeros((pad,), jnp.int32)])
    w = jnp.concatenate([edge_weight, jnp.zeros((pad,), jnp.float32)])
    src3 = src.reshape(NW, NCHUNK, CHUNK)
    dst3 = dst.reshape(NW, NCHUNK, CHUNK)
    w3 = w.reshape(NW, NCHUNK, CHUNK)
    zero_hid = jnp.zeros((N_PAD, D_HID), jnp.float32)
    zero_out = jnp.zeros((N_PAD, D_OUT), jnp.float32)

    h = pl.pallas_call(
        _mm1_body,
        grid=(10,),
        in_specs=[pl.BlockSpec((1000, D_IN), lambda i: (i, 0)),
                  pl.BlockSpec((D_IN, D_HID), lambda i: (0, 0))],
        out_specs=pl.BlockSpec((1000, D_HID), lambda i: (i, 0)),
        out_shape=jax.ShapeDtypeStruct((N_NODES, D_HID), jnp.float32),
    )(x, W1)

    p = _make_spmm(D_HID)(h, src3, dst3, w3, zero_hid)
    p = p.reshape(NC, N_PAD, D_HID)[:, :N_NODES]

    r = pl.pallas_call(
        _relu_body,
        grid=(10,),
        in_specs=[pl.BlockSpec((1000, D_HID), lambda i: (i, 0)),
                  pl.BlockSpec((1000, D_HID), lambda i: (i, 0))],
        out_specs=pl.BlockSpec((1000, D_HID), lambda i: (i, 0)),
        out_shape=jax.ShapeDtypeStruct((N_NODES, D_HID), jnp.float32),
    )(p[0], p[1])

    o = _make_spmm(D_HID)(r, src3, dst3, w3, zero_hid)
    o = o.reshape(NC, N_PAD, D_HID)[:, :N_NODES]

    maskf = mask.astype(jnp.float32).reshape(N_NODES, 1)
    loss, acc = pl.pallas_call(
        _loss_body,
        in_specs=[pl.BlockSpec((N_NODES, D_HID), lambda: (0, 0)),
                  pl.BlockSpec((N_NODES, D_HID), lambda: (0, 0)),
                  pl.BlockSpec((N_NODES, D_OUT), lambda: (0, 0)),
                  pl.BlockSpec((N_NODES, 1), lambda: (0, 0)),
                  pl.BlockSpec((D_IN, D_HID), lambda: (0, 0)),
                  pl.BlockSpec((D_HID, D_OUT), lambda: (0, 0))],
        out_specs=[pl.BlockSpec((1, 1), lambda: (0, 0)),
                   pl.BlockSpec((1, 1), lambda: (0, 0))],
        out_shape=[jax.ShapeDtypeStruct((1, 1), jnp.float32),
                   jax.ShapeDtypeStruct((1, 1), jnp.float32)],
    )(o[0], o[1], label, maskf, W1, W2)
    return (loss[0, 0], acc[0, 0])


# speedup vs baseline: 3.8583x; 1.2721x over previous
import functools

import jax
import jax.numpy as jnp
from jax import lax
from jax.experimental import pallas as pl
from jax.experimental.pallas import tpu as pltpu, tpu_sc as plsc

N_NODES = 10000
N_EDGES = 320000
D_IN = 128
D_HID = 128
D_OUT = 16
WEIGHT_DECAY = 5e-4

NC = 2
NS = 16
NW = NC * NS
E_PAD = 327680
EPW = E_PAD // NW
CHUNK = 64
NCHUNK = EPW // CHUNK
CPS = 32
NSUPER = NCHUNK // CPS
N_PAD = 10240
ROWS_PER_SUB = N_PAD // NS


@functools.lru_cache(maxsize=None)
def _make_spmm(d):
    mesh = plsc.VectorSubcoreMesh(core_axis_name="c", subcore_axis_name="s")

    @functools.partial(
        pl.kernel,
        out_type=jax.ShapeDtypeStruct((NC * N_PAD, d), jnp.float32),
        mesh=mesh,
        scratch_types=[
            pltpu.VMEM_SHARED((N_PAD, d), jnp.float32),
            pltpu.VMEM((CPS, CHUNK), jnp.int32),
            pltpu.VMEM((CPS, CHUNK), jnp.int32),
            pltpu.VMEM((CPS, CHUNK), jnp.float32),
            pltpu.VMEM((CHUNK, d), jnp.float32),
            pltpu.VMEM((CHUNK, d), jnp.float32),
            pltpu.SemaphoreType.DMA,
            pltpu.SemaphoreType.DMA,
        ],
    )
    def spmm(h_hbm, src_hbm, dst_hbm, w_hbm, zero_hbm, out_hbm,
             acc, src_v, dst_v, w_v, rows0_v, rows1_v, sem0, sem1):
        c = lax.axis_index("c")
        s = lax.axis_index("s")
        wid = s * NC + c
        pltpu.sync_copy(zero_hbm.at[pl.ds(s * ROWS_PER_SUB, ROWS_PER_SUB)],
                        acc.at[pl.ds(s * ROWS_PER_SUB, ROWS_PER_SUB)])
        plsc.subcore_barrier()

        def scale(rows_v, k):
            for g in range(CHUNK // 16):
                wvec = w_v[k, pl.ds(g * 16, 16)]
                for j in range(16):
                    e = g * 16 + j
                    wsc = wvec[j]
                    for f in range(d // 16):
                        rows_v[e, pl.ds(f * 16, 16)] = (
                            rows_v[e, pl.ds(f * 16, 16)] * wsc)

        def super_body(u, carry):
            pltpu.sync_copy(src_hbm.at[wid, pl.ds(u * CPS, CPS)], src_v)
            pltpu.sync_copy(dst_hbm.at[wid, pl.ds(u * CPS, CPS)], dst_v)
            pltpu.sync_copy(w_hbm.at[wid, pl.ds(u * CPS, CPS)], w_v)
            pltpu.async_copy(h_hbm.at[src_v.at[0]], rows0_v, sem0)

            def pair_body(j, carry2):
                k0 = 2 * j
                k1 = 2 * j + 1
                cp1 = pltpu.async_copy(h_hbm.at[src_v.at[k1]], rows1_v, sem1)
                pltpu.make_async_copy(h_hbm.at[src_v.at[k0]], rows0_v,
                                      sem0).wait()
                scale(rows0_v, k0)
                pltpu.sync_copy(rows0_v, acc.at[dst_v.at[k0]], add=True)

                @pl.when(j + 1 < CPS // 2)
                def _():
                    pltpu.async_copy(h_hbm.at[src_v.at[k0 + 2]], rows0_v,
                                     sem0)

                cp1.wait()
                scale(rows1_v, k1)
                pltpu.sync_copy(rows1_v, acc.at[dst_v.at[k1]], add=True)
                return carry2

            lax.fori_loop(0, CPS // 2, pair_body, carry)
            return carry

        lax.fori_loop(0, NSUPER, super_body, 0)
        plsc.subcore_barrier()
        pltpu.sync_copy(acc.at[pl.ds(s * ROWS_PER_SUB, ROWS_PER_SUB)],
                        out_hbm.at[pl.ds(c * N_PAD + s * ROWS_PER_SUB,
                                         ROWS_PER_SUB)])

    return spmm


def _mm1_body(x_ref, w_ref, o_ref):
    o_ref[...] = jnp.dot(x_ref[...], w_ref[...],
                         preferred_element_type=jnp.float32)


def _relu_body(p0_ref, p1_ref, o_ref):
    o_ref[...] = jnp.maximum(p0_ref[...] + p1_ref[...], 0.0)


def _loss_body(o0_ref, o1_ref, lab_ref, m_ref, w1_ref, w2_ref, loss_ref,
               acc_ref):
    out = jnp.dot(o0_ref[...] + o1_ref[...], w2_ref[...],
                  preferred_element_type=jnp.float32)
    lab = lab_ref[...]
    m = m_ref[...]
    mx = jnp.max(out, axis=-1, keepdims=True)
    sh = out - mx
    lse = jnp.log(jnp.sum(jnp.exp(sh), axis=-1, keepdims=True))
    ce = -jnp.sum(lab * (sh - lse), axis=-1, keepdims=True)
    iota = lax.broadcasted_iota(jnp.int32, (N_NODES, D_OUT), 1)
    ao = jnp.min(jnp.where(out == mx, iota, D_OUT), axis=-1, keepdims=True)
    lmx = jnp.max(lab, axis=-1, keepdims=True)
    al = jnp.min(jnp.where(lab == lmx, iota, D_OUT), axis=-1, keepdims=True)
    correct = (ao == al).astype(jnp.float32)
    msum = jnp.sum(m)
    wd = WEIGHT_DECAY * 0.5 * jnp.sum(w1_ref[...] * w1_ref[...])
    loss_ref[...] = jnp.reshape(wd + jnp.sum(ce * m) / msum, (1, 1))
    acc_ref[...] = jnp.reshape(jnp.sum(correct * m) / msum, (1, 1))


def kernel(x, label, mask, edge_index, edge_weight, W1, W2):
    src = edge_index[0].astype(jnp.int32)
    dst = edge_index[1].astype(jnp.int32)
    pad = E_PAD - N_EDGES
    src = jnp.concatenate([src, jnp.zeros((pad,), jnp.int32)])
    dst = jnp.concatenate([dst, jnp.zeros((pad,), jnp.int32)])
    w = jnp.concatenate([edge_weight, jnp.zeros((pad,), jnp.float32)])
    src3 = src.reshape(NW, NCHUNK, CHUNK)
    dst3 = dst.reshape(NW, NCHUNK, CHUNK)
    w3 = w.reshape(NW, NCHUNK, CHUNK)
    zero_hid = jnp.zeros((N_PAD, D_HID), jnp.float32)
    zero_out = jnp.zeros((N_PAD, D_OUT), jnp.float32)

    h = pl.pallas_call(
        _mm1_body,
        grid=(10,),
        in_specs=[pl.BlockSpec((1000, D_IN), lambda i: (i, 0)),
                  pl.BlockSpec((D_IN, D_HID), lambda i: (0, 0))],
        out_specs=pl.BlockSpec((1000, D_HID), lambda i: (i, 0)),
        out_shape=jax.ShapeDtypeStruct((N_NODES, D_HID), jnp.float32),
    )(x, W1)

    p = _make_spmm(D_HID)(h, src3, dst3, w3, zero_hid)
    p = p.reshape(NC, N_PAD, D_HID)[:, :N_NODES]

    r = pl.pallas_call(
        _relu_body,
        grid=(10,),
        in_specs=[pl.BlockSpec((1000, D_HID), lambda i: (i, 0)),
                  pl.BlockSpec((1000, D_HID), lambda i: (i, 0))],
        out_specs=pl.BlockSpec((1000, D_HID), lambda i: (i, 0)),
        out_shape=jax.ShapeDtypeStruct((N_NODES, D_HID), jnp.float32),
    )(p[0], p[1])

    o = _make_spmm(D_HID)(r, src3, dst3, w3, zero_hid)
    o = o.reshape(NC, N_PAD, D_HID)[:, :N_NODES]

    maskf = mask.astype(jnp.float32).reshape(N_NODES, 1)
    loss, acc = pl.pallas_call(
        _loss_body,
        in_specs=[pl.BlockSpec((N_NODES, D_HID), lambda: (0, 0)),
                  pl.BlockSpec((N_NODES, D_HID), lambda: (0, 0)),
                  pl.BlockSpec((N_NODES, D_OUT), lambda: (0, 0)),
                  pl.BlockSpec((N_NODES, 1), lambda: (0, 0)),
                  pl.BlockSpec((D_IN, D_HID), lambda: (0, 0)),
                  pl.BlockSpec((D_HID, D_OUT), lambda: (0, 0))],
        out_specs=[pl.BlockSpec((1, 1), lambda: (0, 0)),
                   pl.BlockSpec((1, 1), lambda: (0, 0))],
        out_shape=[jax.ShapeDtypeStruct((1, 1), jnp.float32),
                   jax.ShapeDtypeStruct((1, 1), jnp.float32)],
    )(o[0], o[1], label, maskf, W1, W2)
    return (loss[0, 0], acc[0, 0])
